# Initial kernel scaffold; baseline (speedup 1.0000x reference)
import jax
import jax.numpy as jnp
from jax.experimental import pallas as pl


def kernel(adj_t, x, lin0_W, lin0_b, conv_W, lin1_W, lin1_b):
    raise NotImplementedError("write your pallas kernel here")


# speedup vs baseline: 3.1283x; 3.1283x over previous
import functools

import numpy as np
import jax
import jax.numpy as jnp
from jax import lax
from jax.experimental import pallas as pl
from jax.experimental.pallas import tpu as pltpu
from jax.experimental.pallas import tpu_sc as plsc

N = 10000
D_FEAT = 128
HIDDEN = 256
CLASSES = 40
LAYERS = 4
ALPHA = 0.1
THETA = 0.5
E = 320000

NTILES = 16
CH = 80
EPT = E // NTILES
NCH = EPT // CH
RPT = N // NTILES
ZR = 125
RB = 1000


def _sc_segment_sum(h_flat, src4, dst2):
    mesh = plsc.VectorSubcoreMesh(core_axis_name="c", subcore_axis_name="s")

    @functools.partial(
        pl.kernel,
        mesh=mesh,
        out_type=jax.ShapeDtypeStruct((2, N, 2, 64), jnp.float32),
        scratch_types=[
            pltpu.VMEM((NCH, CH), jnp.int32),
            pltpu.VMEM((NCH, CH), jnp.int32),
            pltpu.VMEM((CH, 64), jnp.float32),
            pltpu.VMEM((ZR, 64), jnp.float32),
            pltpu.VMEM_SHARED((N, 64), jnp.float32),
            pltpu.SemaphoreType.DMA,
        ],
        compiler_params=pltpu.CompilerParams(use_tc_tiling_on_sc=False),
    )
    def k(h_hbm, src_hbm, dst_hbm, out_hbm, src_v, dst_v, rows_v, zero_v, agg_sh, sem):
        c = lax.axis_index("c")
        s = lax.axis_index("s")
        pltpu.sync_copy(dst_hbm.at[s], dst_v)
        z16 = jnp.zeros((16,), jnp.float32)

        def zbody(i, carry):
            for q in range(4):
                zero_v[i, pl.ds(q * 16, 16)] = z16
            return carry

        lax.fori_loop(0, ZR, zbody, 0)

        for phase in range(2):
            pltpu.sync_copy(src_hbm.at[c, phase, s], src_v)
            for j in range(RPT // ZR):
                pltpu.sync_copy(zero_v, agg_sh.at[pl.ds(s * RPT + j * ZR, ZR)])
            plsc.subcore_barrier()

            def ebody(j, carry):
                pltpu.async_copy(h_hbm.at[src_v.at[j]], rows_v, sem).wait()
                pltpu.sync_copy(rows_v, agg_sh.at[dst_v.at[j]], add=True)
                return carry

            lax.fori_loop(0, NCH, ebody, 0)
            plsc.subcore_barrier()
            pltpu.sync_copy(agg_sh.at[pl.ds(s * RPT, RPT)],
                            out_hbm.at[c, pl.ds(s * RPT, RPT), phase])

    return k(h_flat, src4, dst2)


def _lin0(x, w, b):
    def body(x_ref, w_ref, b_ref, o_ref):
        y = jnp.dot(x_ref[...], w_ref[...], preferred_element_type=jnp.float32)
        y = jnp.maximum(y + b_ref[...], 0.0)
        o_ref[0] = y[:, :128]
        o_ref[1] = y[:, 128:]

    return pl.pallas_call(
        body,
        grid=(N // RB,),
        in_specs=[
            pl.BlockSpec((RB, D_FEAT), lambda i: (i, 0)),
            pl.BlockSpec((D_FEAT, HIDDEN), lambda i: (0, 0)),
            pl.BlockSpec((1, HIDDEN), lambda i: (0, 0)),
        ],
        out_specs=pl.BlockSpec((2, RB, 128), lambda i: (0, i, 0)),
        out_shape=jax.ShapeDtypeStruct((2, N, 128), jnp.float32),
    )(x, w, b)


def _layer_tc(agg, h0, w, beta):
    def body(a_ref, h0_ref, w_ref, o_ref):
        hm0 = (1.0 - ALPHA) * a_ref[0] + ALPHA * h0_ref[0]
        hm1 = (1.0 - ALPHA) * a_ref[1] + ALPHA * h0_ref[1]
        hm = jnp.concatenate([hm0, hm1], axis=1)
        y = jnp.dot(hm, w_ref[...], preferred_element_type=jnp.float32)
        y = jnp.maximum((1.0 - beta) * hm + beta * y, 0.0)
        o_ref[0] = y[:, :128]
        o_ref[1] = y[:, 128:]

    return pl.pallas_call(
        body,
        grid=(N // RB,),
        in_specs=[
            pl.BlockSpec((2, RB, 128), lambda i: (0, i, 0)),
            pl.BlockSpec((2, RB, 128), lambda i: (0, i, 0)),
            pl.BlockSpec((HIDDEN, HIDDEN), lambda i: (0, 0)),
        ],
        out_specs=pl.BlockSpec((2, RB, 128), lambda i: (0, i, 0)),
        out_shape=jax.ShapeDtypeStruct((2, N, 128), jnp.float32),
    )(agg, h0, w)


def _final_tc(h, w, b):
    def body(h_ref, w_ref, b_ref, o_ref):
        hm = jnp.concatenate([h_ref[0], h_ref[1]], axis=1)
        y = jnp.dot(hm, w_ref[...], preferred_element_type=jnp.float32) + b_ref[...]
        m = jnp.max(y, axis=1, keepdims=True)
        ls = jnp.log(jnp.sum(jnp.exp(y - m), axis=1, keepdims=True))
        o_ref[...] = y - m - ls

    return pl.pallas_call(
        body,
        grid=(N // RB,),
        in_specs=[
            pl.BlockSpec((2, RB, 128), lambda i: (0, i, 0)),
            pl.BlockSpec((HIDDEN, CLASSES), lambda i: (0, 0)),
            pl.BlockSpec((1, CLASSES), lambda i: (0, 0)),
        ],
        out_specs=pl.BlockSpec((RB, CLASSES), lambda i: (i, 0)),
        out_shape=jax.ShapeDtypeStruct((N, CLASSES), jnp.float32),
    )(h, w, b)


def kernel(adj_t, x, lin0_W, lin0_b, conv_W, lin1_W, lin1_b):
    src = adj_t[0].astype(jnp.int32)
    dst = adj_t[1].astype(jnp.int32)
    offs = jnp.array([0, 1, 2 * N, 2 * N + 1], jnp.int32).reshape(2, 2, 1)
    src4 = (2 * src[None, None, :] + offs).reshape(2, 2, NTILES, NCH, CH)
    dst2 = dst.reshape(NTILES, NCH, CH)

    h = _lin0(x, lin0_W, lin0_b.reshape(1, HIDDEN))
    h0 = h
    for layer in range(LAYERS):
        beta = float(np.log(THETA / (layer + 1) + 1.0))
        agg = _sc_segment_sum(h.reshape(4 * N, 64), src4, dst2)
        agg = agg.reshape(2, N, 128)
        h = _layer_tc(agg, h0, conv_W[layer], beta)
    return _final_tc(h, lin1_W, lin1_b.reshape(1, CLASSES))


# speedup vs baseline: 6.5198x; 2.0842x over previous
import functools

import numpy as np
import jax
import jax.numpy as jnp
from jax import lax
from jax.experimental import pallas as pl
from jax.experimental.pallas import tpu as pltpu
from jax.experimental.pallas import tpu_sc as plsc

N = 10000
D_FEAT = 128
HIDDEN = 256
CLASSES = 40
LAYERS = 4
ALPHA = 0.1
THETA = 0.5
E = 320000

NTILES = 16
CH = 125
EPT = E // NTILES
NCH = EPT // CH
NBUF = 4
RPT = N // NTILES
ZR = 25
RB = 1000


def _sc_segment_sum(h_flat, src4, dst2):
    mesh = plsc.VectorSubcoreMesh(core_axis_name="c", subcore_axis_name="s")

    @functools.partial(
        pl.kernel,
        mesh=mesh,
        out_type=jax.ShapeDtypeStruct((2, N, 2, 64), jnp.float32),
        scratch_types=[
            pltpu.VMEM((NCH, CH), jnp.int32),
            pltpu.VMEM((NCH, CH), jnp.int32),
            pltpu.VMEM((NBUF, CH, 64), jnp.float32),
            pltpu.VMEM((ZR, 64), jnp.float32),
            pltpu.VMEM_SHARED((N, 64), jnp.float32),
            pltpu.SemaphoreType.DMA((NBUF,)),
            pltpu.SemaphoreType.DMA((NBUF,)),
        ],
        compiler_params=pltpu.CompilerParams(use_tc_tiling_on_sc=False),
    )
    def k(h_hbm, src_hbm, dst_hbm, out_hbm, src_v, dst_v, rows_v, zero_v,
          agg_sh, gsem, ssem):
        c = lax.axis_index("c")
        s = lax.axis_index("s")
        pltpu.sync_copy(dst_hbm.at[s], dst_v)
        z16 = jnp.zeros((16,), jnp.float32)

        def zbody(i, carry):
            for q in range(4):
                zero_v[i, pl.ds(q * 16, 16)] = z16
            return carry

        lax.fori_loop(0, ZR, zbody, 0)

        def gather(j, b):
            pltpu.async_copy(h_hbm.at[src_v.at[j]], rows_v.at[b], gsem.at[b])

        def gather_wait(j, b):
            pltpu.make_async_copy(h_hbm.at[src_v.at[j]], rows_v.at[b],
                                  gsem.at[b]).wait()

        def scatter(j, b):
            pltpu.async_copy(rows_v.at[b], agg_sh.at[dst_v.at[j]],
                             ssem.at[b], add=True)

        def scatter_wait(j, b):
            pltpu.make_async_copy(rows_v.at[b], agg_sh.at[dst_v.at[j]],
                                  ssem.at[b]).wait()

        for phase in range(2):
            pltpu.sync_copy(src_hbm.at[c, phase, s], src_v)
            for j in range(RPT // ZR):
                pltpu.sync_copy(zero_v, agg_sh.at[pl.ds(s * RPT + j * ZR, ZR)])
            plsc.subcore_barrier()

            for b in range(NBUF):
                gather(b, b)

            def gbody(g, carry):
                for b in range(NBUF):
                    j = g * NBUF + b
                    gather_wait(j, b)
                    scatter(j, b)
                for b in range(NBUF):
                    j = g * NBUF + b
                    scatter_wait(j, b)

                    @pl.when(j + NBUF < NCH)
                    def _():
                        gather(j + NBUF, b)
                return carry

            lax.fori_loop(0, NCH // NBUF, gbody, 0)
            plsc.subcore_barrier()
            pltpu.sync_copy(agg_sh.at[pl.ds(s * RPT, RPT)],
                            out_hbm.at[c, pl.ds(s * RPT, RPT), phase])

    return k(h_flat, src4, dst2)


def _lin0(x, w, b):
    def body(x_ref, w_ref, b_ref, o_ref):
        y = jnp.dot(x_ref[...], w_ref[...], preferred_element_type=jnp.float32)
        y = jnp.maximum(y + b_ref[...], 0.0)
        o_ref[0] = y[:, :128]
        o_ref[1] = y[:, 128:]

    return pl.pallas_call(
        body,
        grid=(N // RB,),
        in_specs=[
            pl.BlockSpec((RB, D_FEAT), lambda i: (i, 0)),
            pl.BlockSpec((D_FEAT, HIDDEN), lambda i: (0, 0)),
            pl.BlockSpec((1, HIDDEN), lambda i: (0, 0)),
        ],
        out_specs=pl.BlockSpec((2, RB, 128), lambda i: (0, i, 0)),
        out_shape=jax.ShapeDtypeStruct((2, N, 128), jnp.float32),
    )(x, w, b)


def _layer_tc(agg, h0, w, beta):
    def body(a_ref, h0_ref, w_ref, o_ref):
        hm0 = (1.0 - ALPHA) * a_ref[0] + ALPHA * h0_ref[0]
        hm1 = (1.0 - ALPHA) * a_ref[1] + ALPHA * h0_ref[1]
        hm = jnp.concatenate([hm0, hm1], axis=1)
        y = jnp.dot(hm, w_ref[...], preferred_element_type=jnp.float32)
        y = jnp.maximum((1.0 - beta) * hm + beta * y, 0.0)
        o_ref[0] = y[:, :128]
        o_ref[1] = y[:, 128:]

    return pl.pallas_call(
        body,
        grid=(N // RB,),
        in_specs=[
            pl.BlockSpec((2, RB, 128), lambda i: (0, i, 0)),
            pl.BlockSpec((2, RB, 128), lambda i: (0, i, 0)),
            pl.BlockSpec((HIDDEN, HIDDEN), lambda i: (0, 0)),
        ],
        out_specs=pl.BlockSpec((2, RB, 128), lambda i: (0, i, 0)),
        out_shape=jax.ShapeDtypeStruct((2, N, 128), jnp.float32),
    )(agg, h0, w)


def _final_tc(h, w, b):
    def body(h_ref, w_ref, b_ref, o_ref):
        hm = jnp.concatenate([h_ref[0], h_ref[1]], axis=1)
        y = jnp.dot(hm, w_ref[...], preferred_element_type=jnp.float32) + b_ref[...]
        m = jnp.max(y, axis=1, keepdims=True)
        ls = jnp.log(jnp.sum(jnp.exp(y - m), axis=1, keepdims=True))
        o_ref[...] = y - m - ls

    return pl.pallas_call(
        body,
        grid=(N // RB,),
        in_specs=[
            pl.BlockSpec((2, RB, 128), lambda i: (0, i, 0)),
            pl.BlockSpec((HIDDEN, CLASSES), lambda i: (0, 0)),
            pl.BlockSpec((1, CLASSES), lambda i: (0, 0)),
        ],
        out_specs=pl.BlockSpec((RB, CLASSES), lambda i: (i, 0)),
        out_shape=jax.ShapeDtypeStruct((N, CLASSES), jnp.float32),
    )(h, w, b)


def kernel(adj_t, x, lin0_W, lin0_b, conv_W, lin1_W, lin1_b):
    src = adj_t[0].astype(jnp.int32)
    dst = adj_t[1].astype(jnp.int32)
    offs = jnp.array([0, 1, 2 * N, 2 * N + 1], jnp.int32).reshape(2, 2, 1)
    src4 = (2 * src[None, None, :] + offs).reshape(2, 2, NTILES, NCH, CH)
    dst2 = dst.reshape(NTILES, NCH, CH)

    h = _lin0(x, lin0_W, lin0_b.reshape(1, HIDDEN))
    h0 = h
    for layer in range(LAYERS):
        beta = float(np.log(THETA / (layer + 1) + 1.0))
        agg = _sc_segment_sum(h.reshape(4 * N, 64), src4, dst2)
        agg = agg.reshape(2, N, 128)
        h = _layer_tc(agg, h0, conv_W[layer], beta)
    return _final_tc(h, lin1_W, lin1_b.reshape(1, CLASSES))


# speedup vs baseline: 7.3586x; 1.1287x over previous
import functools

import numpy as np
import jax
import jax.numpy as jnp
from jax import lax
from jax.experimental import pallas as pl
from jax.experimental.pallas import tpu as pltpu
from jax.experimental.pallas import tpu_sc as plsc

N = 10000
D_FEAT = 128
HIDDEN = 256
CLASSES = 40
LAYERS = 4
ALPHA = 0.1
THETA = 0.5
E = 320000

NTILES = 16
CH = 125
EPT = E // NTILES
NCH = EPT // CH
NBUF = 4
RPT = N // NTILES
RB = 1000


def _sc_segment_sum(h_q, src_r, dst_r):
    mesh = plsc.VectorSubcoreMesh(core_axis_name="c", subcore_axis_name="s")

    @functools.partial(
        pl.kernel,
        mesh=mesh,
        out_type=jax.ShapeDtypeStruct((4, N, 64), jnp.float32),
        scratch_types=[
            pltpu.VMEM((NCH, CH), jnp.int32),
            pltpu.VMEM((NCH, CH), jnp.int32),
            pltpu.VMEM((NBUF, CH, 64), jnp.float32),
            pltpu.VMEM_SHARED((N, 64), jnp.float32),
            pltpu.SemaphoreType.DMA((NBUF,)),
            pltpu.SemaphoreType.DMA((NBUF,)),
        ],
        compiler_params=pltpu.CompilerParams(use_tc_tiling_on_sc=False),
    )
    def k(h_hbm, src_hbm, dst_hbm, out_hbm, src_v, dst_v, rows_v, agg_sh,
          gsem, ssem):
        c = lax.axis_index("c")
        s = lax.axis_index("s")
        pltpu.sync_copy(src_hbm.at[s], src_v)
        pltpu.sync_copy(dst_hbm.at[s], dst_v)

        def scatter(j, b):
            pltpu.async_copy(rows_v.at[b], agg_sh.at[dst_v.at[j]],
                             ssem.at[b], add=True)

        def scatter_wait(j, b):
            pltpu.make_async_copy(rows_v.at[b], agg_sh.at[dst_v.at[j]],
                                  ssem.at[b]).wait()

        z16 = jnp.zeros((16,), jnp.float32)

        for phase in range(2):
            q = 2 * c + phase

            def gather(j, b):
                pltpu.async_copy(h_hbm.at[q].at[src_v.at[j]],
                                 rows_v.at[b], gsem.at[b])

            def gather_wait(j, b):
                pltpu.make_async_copy(h_hbm.at[q].at[src_v.at[j]],
                                      rows_v.at[b], gsem.at[b]).wait()

            def zbody(i, carry):
                for b in range(NBUF):
                    for q in range(4):
                        rows_v[b, i, pl.ds(q * 16, 16)] = z16
                return carry

            lax.fori_loop(0, CH, zbody, 0)
            for j in range(RPT // CH):
                pltpu.sync_copy(rows_v.at[j % NBUF],
                                agg_sh.at[pl.ds(s * RPT + j * CH, CH)])
            plsc.subcore_barrier()

            for b in range(NBUF):
                gather(b, b)

            def gbody(g, carry):
                for b in range(NBUF):
                    j = g * NBUF + b
                    gather_wait(j, b)
                    scatter(j, b)
                for b in range(NBUF):
                    j = g * NBUF + b
                    scatter_wait(j, b)

                    @pl.when(j + NBUF < NCH)
                    def _():
                        gather(j + NBUF, b)
                return carry

            lax.fori_loop(0, NCH // NBUF, gbody, 0)
            plsc.subcore_barrier()
            pltpu.sync_copy(agg_sh.at[pl.ds(s * RPT, RPT)],
                            out_hbm.at[2 * c + phase, pl.ds(s * RPT, RPT)])

    return k(h_q, src_r, dst_r)


def _split_q(o_ref, y):
    for q in range(4):
        o_ref[q] = y[:, 64 * q:64 * (q + 1)]


def _cat_q(ref):
    return jnp.concatenate([ref[q] for q in range(4)], axis=1)


def _lin0(x, w, b):
    def body(x_ref, w_ref, b_ref, o_ref):
        y = jnp.dot(x_ref[...], w_ref[...], preferred_element_type=jnp.float32)
        _split_q(o_ref, jnp.maximum(y + b_ref[...], 0.0))

    return pl.pallas_call(
        body,
        grid=(N // RB,),
        in_specs=[
            pl.BlockSpec((RB, D_FEAT), lambda i: (i, 0)),
            pl.BlockSpec((D_FEAT, HIDDEN), lambda i: (0, 0)),
            pl.BlockSpec((1, HIDDEN), lambda i: (0, 0)),
        ],
        out_specs=pl.BlockSpec((4, RB, 64), lambda i: (0, i, 0)),
        out_shape=jax.ShapeDtypeStruct((4, N, 64), jnp.float32),
    )(x, w, b)


def _layer_tc(agg, h0, w, beta):
    def body(a_ref, h0_ref, w_ref, o_ref):
        hm = (1.0 - ALPHA) * _cat_q(a_ref) + ALPHA * _cat_q(h0_ref)
        y = jnp.dot(hm, w_ref[...], preferred_element_type=jnp.float32)
        _split_q(o_ref, jnp.maximum((1.0 - beta) * hm + beta * y, 0.0))

    return pl.pallas_call(
        body,
        grid=(N // RB,),
        in_specs=[
            pl.BlockSpec((4, RB, 64), lambda i: (0, i, 0)),
            pl.BlockSpec((4, RB, 64), lambda i: (0, i, 0)),
            pl.BlockSpec((HIDDEN, HIDDEN), lambda i: (0, 0)),
        ],
        out_specs=pl.BlockSpec((4, RB, 64), lambda i: (0, i, 0)),
        out_shape=jax.ShapeDtypeStruct((4, N, 64), jnp.float32),
    )(agg, h0, w)


def _last_layer_tc(agg, h0, w, beta, w1, b1):
    def body(a_ref, h0_ref, w_ref, w1_ref, b1_ref, o_ref):
        hm = (1.0 - ALPHA) * _cat_q(a_ref) + ALPHA * _cat_q(h0_ref)
        y = jnp.dot(hm, w_ref[...], preferred_element_type=jnp.float32)
        h = jnp.maximum((1.0 - beta) * hm + beta * y, 0.0)
        z = jnp.dot(h, w1_ref[...], preferred_element_type=jnp.float32)
        z = z + b1_ref[...]
        m = jnp.max(z, axis=1, keepdims=True)
        ls = jnp.log(jnp.sum(jnp.exp(z - m), axis=1, keepdims=True))
        o_ref[...] = z - m - ls

    return pl.pallas_call(
        body,
        grid=(N // RB,),
        in_specs=[
            pl.BlockSpec((4, RB, 64), lambda i: (0, i, 0)),
            pl.BlockSpec((4, RB, 64), lambda i: (0, i, 0)),
            pl.BlockSpec((HIDDEN, HIDDEN), lambda i: (0, 0)),
            pl.BlockSpec((HIDDEN, CLASSES), lambda i: (0, 0)),
            pl.BlockSpec((1, CLASSES), lambda i: (0, 0)),
        ],
        out_specs=pl.BlockSpec((RB, CLASSES), lambda i: (i, 0)),
        out_shape=jax.ShapeDtypeStruct((N, CLASSES), jnp.float32),
    )(agg, h0, w, w1, b1)


def kernel(adj_t, x, lin0_W, lin0_b, conv_W, lin1_W, lin1_b):
    src_r = adj_t[0].astype(jnp.int32).reshape(NTILES, NCH, CH)
    dst_r = adj_t[1].astype(jnp.int32).reshape(NTILES, NCH, CH)

    h = _lin0(x, lin0_W, lin0_b.reshape(1, HIDDEN))
    h0 = h
    for layer in range(LAYERS):
        beta = float(np.log(THETA / (layer + 1) + 1.0))
        agg = _sc_segment_sum(h, src_r, dst_r)
        if layer < LAYERS - 1:
            h = _layer_tc(agg, h0, conv_W[layer], beta)
        else:
            return _last_layer_tc(agg, h0, conv_W[layer], beta,
                                  lin1_W, lin1_b.reshape(1, CLASSES))


# speedup vs baseline: 9.2286x; 1.2541x over previous
import functools

import numpy as np
import jax
import jax.numpy as jnp
from jax import lax
from jax.experimental import pallas as pl
from jax.experimental.pallas import tpu as pltpu
from jax.experimental.pallas import tpu_sc as plsc

N = 10000
D_FEAT = 128
HIDDEN = 256
CLASSES = 40
LAYERS = 4
ALPHA = 0.1
THETA = 0.5
E = 320000

NTILES = 16
CH = 125
EPT = E // NTILES
NCH = EPT // CH
G = 8
NGRP = NCH // G
RPT = N // NTILES
RB = 1000


def _sc_segment_sum(h, src_r, dst_r):
    mesh = plsc.VectorSubcoreMesh(core_axis_name="c", subcore_axis_name="s")

    @functools.partial(
        pl.kernel,
        mesh=mesh,
        out_type=jax.ShapeDtypeStruct((2, N, 128), jnp.float32),
        scratch_types=[
            pltpu.VMEM((2, G, CH), jnp.int32),
            pltpu.VMEM((2, G, CH), jnp.int32),
            pltpu.VMEM((2, CH, 128), jnp.float32),
            pltpu.VMEM_SHARED((N, 128), jnp.float32),
            pltpu.SemaphoreType.DMA((2,)),
            pltpu.SemaphoreType.DMA((2,)),
            pltpu.SemaphoreType.DMA((2,)),
        ],
        compiler_params=pltpu.CompilerParams(use_tc_tiling_on_sc=False),
    )
    def k(h_hbm, src_hbm, dst_hbm, out_hbm, src_i, dst_i, rows_v, agg_sh,
          gsem, ssem, isem):
        c = lax.axis_index("c")
        s = lax.axis_index("s")

        def gather(gg, i, b):
            pltpu.async_copy(h_hbm.at[c].at[src_i.at[gg % 2, i]],
                             rows_v.at[b], gsem.at[b])

        def gather_wait(gg, i, b):
            pltpu.make_async_copy(h_hbm.at[c].at[src_i.at[gg % 2, i]],
                                  rows_v.at[b], gsem.at[b]).wait()

        def scatter(gg, i, b):
            pltpu.async_copy(rows_v.at[b], agg_sh.at[dst_i.at[gg % 2, i]],
                             ssem.at[b], add=True)

        def scatter_wait(gg, i, b):
            pltpu.make_async_copy(rows_v.at[b], agg_sh.at[dst_i.at[gg % 2, i]],
                                  ssem.at[b]).wait()

        def idx_fetch(gg):
            pltpu.async_copy(src_hbm.at[s, gg], src_i.at[gg % 2],
                             isem.at[gg % 2])
            pltpu.async_copy(dst_hbm.at[s, gg], dst_i.at[gg % 2],
                             isem.at[gg % 2])

        def idx_wait(gg):
            pltpu.make_async_copy(src_hbm.at[s, gg], src_i.at[gg % 2],
                                  isem.at[gg % 2]).wait()
            pltpu.make_async_copy(dst_hbm.at[s, gg], dst_i.at[gg % 2],
                                  isem.at[gg % 2]).wait()

        z16 = jnp.zeros((16,), jnp.float32)

        def zbody(i, carry):
            for q in range(8):
                rows_v[0, i, pl.ds(q * 16, 16)] = z16
            return carry

        lax.fori_loop(0, CH, zbody, 0)
        for j in range(RPT // CH):
            pltpu.sync_copy(rows_v.at[0],
                            agg_sh.at[pl.ds(s * RPT + j * CH, CH)])
        pltpu.sync_copy(src_hbm.at[s, 0], src_i.at[0])
        pltpu.sync_copy(dst_hbm.at[s, 0], dst_i.at[0])
        gather(0, 0, 0)
        gather(0, 1, 1)
        plsc.subcore_barrier()

        def gbody(g, carry):
            @pl.when(g + 1 < NGRP)
            def _():
                idx_fetch(g + 1)

            for i in range(G):
                b = i % 2
                gather_wait(g, i, b)
                scatter(g, i, b)
                if i == G - 2:
                    @pl.when(g + 1 < NGRP)
                    def _():
                        idx_wait(g + 1)
                scatter_wait(g, i, b)
                if i + 2 < G:
                    gather(g, i + 2, b)
                else:
                    @pl.when(g + 1 < NGRP)
                    def _():
                        gather(g + 1, i + 2 - G, b)
            return carry

        lax.fori_loop(0, NGRP, gbody, 0)
        plsc.subcore_barrier()
        pltpu.sync_copy(agg_sh.at[pl.ds(s * RPT, RPT)],
                        out_hbm.at[c, pl.ds(s * RPT, RPT)])

    return k(h, src_r, dst_r)


def _split_h(o_ref, y):
    o_ref[0] = y[:, :128]
    o_ref[1] = y[:, 128:]


def _cat_h(ref):
    return jnp.concatenate([ref[0], ref[1]], axis=1)


def _lin0(x, w, b):
    def body(x_ref, w_ref, b_ref, o_ref):
        y = jnp.dot(x_ref[...], w_ref[...], preferred_element_type=jnp.float32)
        _split_h(o_ref, jnp.maximum(y + b_ref[...], 0.0))

    return pl.pallas_call(
        body,
        grid=(N // RB,),
        in_specs=[
            pl.BlockSpec((RB, D_FEAT), lambda i: (i, 0)),
            pl.BlockSpec((D_FEAT, HIDDEN), lambda i: (0, 0)),
            pl.BlockSpec((1, HIDDEN), lambda i: (0, 0)),
        ],
        out_specs=pl.BlockSpec((2, RB, 128), lambda i: (0, i, 0)),
        out_shape=jax.ShapeDtypeStruct((2, N, 128), jnp.float32),
    )(x, w, b)


def _layer_tc(agg, h0, w, beta):
    def body(a_ref, h0_ref, w_ref, o_ref):
        hm = (1.0 - ALPHA) * _cat_h(a_ref) + ALPHA * _cat_h(h0_ref)
        y = jnp.dot(hm, w_ref[...], preferred_element_type=jnp.float32)
        _split_h(o_ref, jnp.maximum((1.0 - beta) * hm + beta * y, 0.0))

    return pl.pallas_call(
        body,
        grid=(N // RB,),
        in_specs=[
            pl.BlockSpec((2, RB, 128), lambda i: (0, i, 0)),
            pl.BlockSpec((2, RB, 128), lambda i: (0, i, 0)),
            pl.BlockSpec((HIDDEN, HIDDEN), lambda i: (0, 0)),
        ],
        out_specs=pl.BlockSpec((2, RB, 128), lambda i: (0, i, 0)),
        out_shape=jax.ShapeDtypeStruct((2, N, 128), jnp.float32),
    )(agg, h0, w)


def _last_layer_tc(agg, h0, w, beta, w1, b1):
    def body(a_ref, h0_ref, w_ref, w1_ref, b1_ref, o_ref):
        hm = (1.0 - ALPHA) * _cat_h(a_ref) + ALPHA * _cat_h(h0_ref)
        y = jnp.dot(hm, w_ref[...], preferred_element_type=jnp.float32)
        hh = jnp.maximum((1.0 - beta) * hm + beta * y, 0.0)
        z = jnp.dot(hh, w1_ref[...], preferred_element_type=jnp.float32)
        z = z + b1_ref[...]
        m = jnp.max(z, axis=1, keepdims=True)
        ls = jnp.log(jnp.sum(jnp.exp(z - m), axis=1, keepdims=True))
        o_ref[...] = z - m - ls

    return pl.pallas_call(
        body,
        grid=(N // RB,),
        in_specs=[
            pl.BlockSpec((2, RB, 128), lambda i: (0, i, 0)),
            pl.BlockSpec((2, RB, 128), lambda i: (0, i, 0)),
            pl.BlockSpec((HIDDEN, HIDDEN), lambda i: (0, 0)),
            pl.BlockSpec((HIDDEN, CLASSES), lambda i: (0, 0)),
            pl.BlockSpec((1, CLASSES), lambda i: (0, 0)),
        ],
        out_specs=pl.BlockSpec((RB, CLASSES), lambda i: (i, 0)),
        out_shape=jax.ShapeDtypeStruct((N, CLASSES), jnp.float32),
    )(agg, h0, w, w1, b1)


def kernel(adj_t, x, lin0_W, lin0_b, conv_W, lin1_W, lin1_b):
    src_r = adj_t[0].astype(jnp.int32).reshape(NTILES, NGRP, G, CH)
    dst_r = adj_t[1].astype(jnp.int32).reshape(NTILES, NGRP, G, CH)

    h = _lin0(x, lin0_W, lin0_b.reshape(1, HIDDEN))
    h0 = h
    for layer in range(LAYERS):
        beta = float(np.log(THETA / (layer + 1) + 1.0))
        agg = _sc_segment_sum(h, src_r, dst_r)
        if layer < LAYERS - 1:
            h = _layer_tc(agg, h0, conv_W[layer], beta)
        else:
            return _last_layer_tc(agg, h0, conv_W[layer], beta,
                                  lin1_W, lin1_b.reshape(1, CLASSES))


# speedup vs baseline: 9.4527x; 1.0243x over previous
import functools

import numpy as np
import jax
import jax.numpy as jnp
from jax import lax
from jax.experimental import pallas as pl
from jax.experimental.pallas import tpu as pltpu
from jax.experimental.pallas import tpu_sc as plsc

N = 10000
D_FEAT = 128
HIDDEN = 256
CLASSES = 40
LAYERS = 4
ALPHA = 0.1
THETA = 0.5
E = 320000

NTILES = 16
CH = 125
EPT = E // NTILES
NCH = EPT // CH
G = 8
NGRP = NCH // G
RPT = N // NTILES
RB = 1000


def _sc_segment_sum(h, adj_r):
    mesh = plsc.VectorSubcoreMesh(core_axis_name="c", subcore_axis_name="s")

    @functools.partial(
        pl.kernel,
        mesh=mesh,
        out_type=jax.ShapeDtypeStruct((2, N, 128), jnp.float32),
        scratch_types=[
            pltpu.VMEM((2, G, CH), jnp.int32),
            pltpu.VMEM((2, G, CH), jnp.int32),
            pltpu.VMEM((2, CH, 128), jnp.float32),
            pltpu.VMEM_SHARED((N, 128), jnp.float32),
            pltpu.SemaphoreType.DMA((2,)),
            pltpu.SemaphoreType.DMA((2,)),
            pltpu.SemaphoreType.DMA((2,)),
        ],
        compiler_params=pltpu.CompilerParams(use_tc_tiling_on_sc=False),
    )
    def k(h_hbm, adj_hbm, out_hbm, src_i, dst_i, rows_v, agg_sh,
          gsem, ssem, isem):
        c = lax.axis_index("c")
        s = lax.axis_index("s")

        def gather(gg, i, b):
            pltpu.async_copy(h_hbm.at[c].at[src_i.at[gg % 2, i]],
                             rows_v.at[b], gsem.at[b])

        def gather_wait(gg, i, b):
            pltpu.make_async_copy(h_hbm.at[c].at[src_i.at[gg % 2, i]],
                                  rows_v.at[b], gsem.at[b]).wait()

        def scatter(gg, i, b):
            pltpu.async_copy(rows_v.at[b], agg_sh.at[dst_i.at[gg % 2, i]],
                             ssem.at[b], add=True)

        def scatter_wait(gg, i, b):
            pltpu.make_async_copy(rows_v.at[b], agg_sh.at[dst_i.at[gg % 2, i]],
                                  ssem.at[b]).wait()

        def idx_fetch(gg):
            pltpu.async_copy(adj_hbm.at[0, s, gg], src_i.at[gg % 2],
                             isem.at[gg % 2])
            pltpu.async_copy(adj_hbm.at[1, s, gg], dst_i.at[gg % 2],
                             isem.at[gg % 2])

        def idx_wait(gg):
            pltpu.make_async_copy(adj_hbm.at[0, s, gg], src_i.at[gg % 2],
                                  isem.at[gg % 2]).wait()
            pltpu.make_async_copy(adj_hbm.at[1, s, gg], dst_i.at[gg % 2],
                                  isem.at[gg % 2]).wait()

        z16 = jnp.zeros((16,), jnp.float32)

        def zbody(i, carry):
            for q in range(8):
                rows_v[0, i, pl.ds(q * 16, 16)] = z16
            return carry

        lax.fori_loop(0, CH, zbody, 0)
        for j in range(RPT // CH):
            pltpu.sync_copy(rows_v.at[0],
                            agg_sh.at[pl.ds(s * RPT + j * CH, CH)])
        pltpu.sync_copy(adj_hbm.at[0, s, 0], src_i.at[0])
        pltpu.sync_copy(adj_hbm.at[1, s, 0], dst_i.at[0])
        gather(0, 0, 0)
        gather(0, 1, 1)
        plsc.subcore_barrier()

        def gbody(g, carry):
            @pl.when(g + 1 < NGRP)
            def _():
                idx_fetch(g + 1)

            for i in range(G):
                b = i % 2
                gather_wait(g, i, b)
                scatter(g, i, b)
                if i == G - 2:
                    @pl.when(g + 1 < NGRP)
                    def _():
                        idx_wait(g + 1)
                scatter_wait(g, i, b)
                if i + 2 < G:
                    gather(g, i + 2, b)
                else:
                    @pl.when(g + 1 < NGRP)
                    def _():
                        gather(g + 1, i + 2 - G, b)
            return carry

        lax.fori_loop(0, NGRP, gbody, 0)
        plsc.subcore_barrier()
        pltpu.sync_copy(agg_sh.at[pl.ds(s * RPT, RPT)],
                        out_hbm.at[c, pl.ds(s * RPT, RPT)])

    return k(h, adj_r)


def _split_h(o_ref, y):
    o_ref[0] = y[:, :128]
    o_ref[1] = y[:, 128:]


def _cat_h(ref):
    return jnp.concatenate([ref[0], ref[1]], axis=1)


def _lin0(x, w, b):
    def body(x_ref, w_ref, b_ref, o_ref):
        y = jnp.dot(x_ref[...], w_ref[...], preferred_element_type=jnp.float32)
        _split_h(o_ref, jnp.maximum(y + b_ref[...], 0.0))

    return pl.pallas_call(
        body,
        grid=(N // RB,),
        in_specs=[
            pl.BlockSpec((RB, D_FEAT), lambda i: (i, 0)),
            pl.BlockSpec((D_FEAT, HIDDEN), lambda i: (0, 0)),
            pl.BlockSpec((1, HIDDEN), lambda i: (0, 0)),
        ],
        out_specs=pl.BlockSpec((2, RB, 128), lambda i: (0, i, 0)),
        out_shape=jax.ShapeDtypeStruct((2, N, 128), jnp.float32),
    )(x, w, b)


def _layer_tc(agg, h0, w, beta):
    def body(a_ref, h0_ref, w_ref, o_ref):
        hm = (1.0 - ALPHA) * _cat_h(a_ref) + ALPHA * _cat_h(h0_ref)
        y = jnp.dot(hm, w_ref[...], preferred_element_type=jnp.float32)
        _split_h(o_ref, jnp.maximum((1.0 - beta) * hm + beta * y, 0.0))

    return pl.pallas_call(
        body,
        grid=(N // RB,),
        in_specs=[
            pl.BlockSpec((2, RB, 128), lambda i: (0, i, 0)),
            pl.BlockSpec((2, RB, 128), lambda i: (0, i, 0)),
            pl.BlockSpec((HIDDEN, HIDDEN), lambda i: (0, 0)),
        ],
        out_specs=pl.BlockSpec((2, RB, 128), lambda i: (0, i, 0)),
        out_shape=jax.ShapeDtypeStruct((2, N, 128), jnp.float32),
    )(agg, h0, w)


def _last_layer_tc(agg, h0, w, beta, w1, b1):
    def body(a_ref, h0_ref, w_ref, w1_ref, b1_ref, o_ref):
        hm = (1.0 - ALPHA) * _cat_h(a_ref) + ALPHA * _cat_h(h0_ref)
        y = jnp.dot(hm, w_ref[...], preferred_element_type=jnp.float32)
        hh = jnp.maximum((1.0 - beta) * hm + beta * y, 0.0)
        z = jnp.dot(hh, w1_ref[...], preferred_element_type=jnp.float32)
        z = z + b1_ref[...]
        m = jnp.max(z, axis=1, keepdims=True)
        ls = jnp.log(jnp.sum(jnp.exp(z - m), axis=1, keepdims=True))
        o_ref[...] = z - m - ls

    return pl.pallas_call(
        body,
        grid=(N // RB,),
        in_specs=[
            pl.BlockSpec((2, RB, 128), lambda i: (0, i, 0)),
            pl.BlockSpec((2, RB, 128), lambda i: (0, i, 0)),
            pl.BlockSpec((HIDDEN, HIDDEN), lambda i: (0, 0)),
            pl.BlockSpec((HIDDEN, CLASSES), lambda i: (0, 0)),
            pl.BlockSpec((1, CLASSES), lambda i: (0, 0)),
        ],
        out_specs=pl.BlockSpec((RB, CLASSES), lambda i: (i, 0)),
        out_shape=jax.ShapeDtypeStruct((N, CLASSES), jnp.float32),
    )(agg, h0, w, w1, b1)


def kernel(adj_t, x, lin0_W, lin0_b, conv_W, lin1_W, lin1_b):
    adj_r = adj_t.astype(jnp.int32).reshape(2, NTILES, NGRP, G, CH)

    h = _lin0(x, lin0_W, lin0_b.reshape(1, HIDDEN))
    h0 = h
    for layer in range(LAYERS):
        beta = float(np.log(THETA / (layer + 1) + 1.0))
        agg = _sc_segment_sum(h, adj_r)
        if layer < LAYERS - 1:
            h = _layer_tc(agg, h0, conv_W[layer], beta)
        else:
            return _last_layer_tc(agg, h0, conv_W[layer], beta,
                                  lin1_W, lin1_b.reshape(1, CLASSES))
